# baseline (speedup 1.0000x reference)
import functools

import jax
import jax.numpy as jnp
from jax import lax
from jax.experimental import pallas as pl
from jax.experimental.pallas import tpu as pltpu
from jax.experimental.pallas import tpu_sc as plsc

B, L = 4096, 200
D_A, D_C = 64, 32
D_OUT = D_A + D_C
BL = B * L
NV = 1000

NC, NS = 2, 16
NW = NC * NS
CH = 128
PER_W = BL // NW
NITER = PER_W // CH
NBUF = 4
HEAD = 4
TAIL = 4
assert (NITER - HEAD - TAIL) % NBUF == 0


def _emb_body(aidx_hbm, cidx_hbm, emb_hbm, chem_hbm, out_hbm,
              aidx_v, cidx_v, obuf, emb_sp, chem_sp, g1sems, g2sems, wsems):
    sid = lax.axis_index("s")
    wid = sid * NC + lax.axis_index("c")
    row0 = wid * PER_W
    it0 = wid * NITER

    @pl.when(sid == 0)
    def _stage_tables():
        pltpu.sync_copy(emb_hbm, emb_sp)
        pltpu.sync_copy(chem_hbm, chem_sp)

    pltpu.sync_copy(aidx_hbm.at[pl.ds(it0, NITER)], aidx_v)
    pltpu.sync_copy(cidx_hbm.at[pl.ds(it0, NITER)], cidx_v)
    plsc.subcore_barrier()

    def g1_start(j, b):
        pltpu.async_copy(emb_sp.at[aidx_v.at[j]], obuf.at[b],
                         g1sems.at[b])

    def g1_wait(b):
        pltpu.make_async_copy(emb_sp.at[aidx_v.at[0]], obuf.at[b],
                              g1sems.at[b]).wait()

    def g2_start(j, b):
        pltpu.async_copy(chem_sp.at[cidx_v.at[j]], obuf.at[b],
                         g2sems.at[b], add=True)

    def g2_wait(b):
        pltpu.make_async_copy(chem_sp.at[cidx_v.at[0]], obuf.at[b],
                              g2sems.at[b]).wait()

    def w_start(j, b):
        pltpu.async_copy(obuf.at[b], out_hbm.at[pl.ds(row0 + j * CH, CH)],
                         wsems.at[b])

    def w_wait(b):
        pltpu.make_async_copy(obuf.at[b], out_hbm.at[pl.ds(row0, CH)],
                              wsems.at[b]).wait()

    def step(j):
        jp = j + 2
        if jp < NITER:
            if j - (NBUF - 2) >= 0:
                w_wait(jp % NBUF)
            g1_start(jp, jp % NBUF)
        jn = j + 1
        if 0 <= jn < NITER:
            g1_wait(jn % NBUF)
            g2_start(jn, jn % NBUF)
        if j >= 0:
            g2_wait(j % NBUF)
            w_start(j, j % NBUF)

    for j in range(-2, HEAD):
        step(j)

    @pl.loop(HEAD, NITER - TAIL, step=NBUF)
    def _main(g):
        for b in range(NBUF):
            j = g + b
            w_wait((j + 2) % NBUF)
            g1_start(j + 2, (j + 2) % NBUF)
            g1_wait((j + 1) % NBUF)
            g2_start(j + 1, (j + 1) % NBUF)
            g2_wait(j % NBUF)
            w_start(j, j % NBUF)

    for j in range(NITER - TAIL, NITER):
        step(j)

    for j in range(NITER - NBUF, NITER):
        w_wait(j % NBUF)


_emb_lookup = functools.partial(
    pl.kernel,
    out_type=jax.ShapeDtypeStruct((BL, D_OUT), jnp.float32),
    mesh=plsc.VectorSubcoreMesh(core_axis_name="c", subcore_axis_name="s",
                                num_cores=NC, num_subcores=NS),
    scratch_types=[
        pltpu.VMEM((NITER, CH), jnp.int32),
        pltpu.VMEM((NITER, CH), jnp.int32),
        pltpu.VMEM((NBUF, CH, D_OUT), jnp.float32),
        pltpu.VMEM_SHARED((NV, D_OUT), jnp.float32),
        pltpu.VMEM_SHARED((NV, D_OUT), jnp.float32),
        pltpu.SemaphoreType.DMA((NBUF,)),
        pltpu.SemaphoreType.DMA((NBUF,)),
        pltpu.SemaphoreType.DMA((NBUF,)),
    ],
)(_emb_body)


def kernel(atom_types, chemistry_types, emb_table, chem_table):
    a = atom_types.reshape(BL // CH, CH).astype(jnp.int32)
    c = chemistry_types.reshape(BL // CH, CH).astype(jnp.int32)
    emb_pad = jnp.pad(emb_table, ((0, 0), (0, D_C)))
    chem_pad = jnp.pad(chem_table, ((0, 0), (D_A, 0)))
    out = _emb_lookup(a, c, emb_pad, chem_pad)
    return out.reshape(B, L, D_OUT)
